# baseline (speedup 1.0000x reference)
import functools

import jax
import jax.numpy as jnp
from jax import lax
from jax.experimental import pallas as pl
from jax.experimental.pallas import tpu as pltpu
from jax.experimental.pallas import tpu_sc as plsc

_NWORKERS = 32
_CHUNK = 16384
_NBUF = 4


def _sc_copy_body(nchunk, src, dst, buf, *sems):
    sem_in = sems[:_NBUF]
    sem_out = sems[_NBUF:]
    per_w = nchunk * _CHUNK
    wid = lax.axis_index("s") * 2 + lax.axis_index("c")
    base = wid * per_w

    def start_load(b, off):
        pltpu.make_async_copy(
            src.at[pl.ds(off, _CHUNK)], buf.at[b], sem_in[b]).start()

    def wait_load(b):
        pltpu.make_async_copy(
            src.at[pl.ds(0, _CHUNK)], buf.at[b], sem_in[b]).wait()

    def start_store(b, off):
        pltpu.make_async_copy(
            buf.at[b], dst.at[pl.ds(off, _CHUNK)], sem_out[b]).start()

    def wait_store(b):
        pltpu.make_async_copy(
            buf.at[0], dst.at[pl.ds(0, _CHUNK)], sem_out[b]).wait()

    for b in range(_NBUF):
        start_load(b, base + b * _CHUNK)

    niter = nchunk // _NBUF

    def body(i, _):
        for b in range(_NBUF):
            wait_load(b)
            start_store(b, base + (i * _NBUF + b) * _CHUNK)
        for b in range(_NBUF):
            wait_store(b)

            @pl.when(i < niter - 1)
            def _():
                start_load(b, base + ((i + 1) * _NBUF + b) * _CHUNK)

        return 0

    lax.fori_loop(0, niter, body, 0)


def _sc_copy(flat):
    (n,) = flat.shape
    per_w = n // _NWORKERS
    nchunk = per_w // _CHUNK
    mesh = plsc.VectorSubcoreMesh(core_axis_name="c", subcore_axis_name="s")
    return pl.kernel(
        functools.partial(_sc_copy_body, nchunk),
        out_type=jax.ShapeDtypeStruct((n,), flat.dtype),
        mesh=mesh,
        scratch_types=(
            [pltpu.VMEM((_NBUF, _CHUNK), flat.dtype)]
            + [pltpu.SemaphoreType.DMA] * (2 * _NBUF)
        ),
    )(flat)


def _tc_copy_body(src_ref, dst_ref):
    dst_ref[...] = src_ref[...]


def _tc_copy(bank):
    dim, size = bank.shape
    blk = 16384
    return pl.pallas_call(
        _tc_copy_body,
        grid=(size // blk,),
        in_specs=[pl.BlockSpec((dim, blk), lambda i: (0, i))],
        out_specs=pl.BlockSpec((dim, blk), lambda i: (0, i)),
        out_shape=jax.ShapeDtypeStruct(bank.shape, bank.dtype),
    )(bank)


def kernel(output, bank):
    tc = _tc_copy(bank)
    sc = _sc_copy(bank.reshape(bank.shape[0] * bank.shape[1]))
    out2 = output + sc[0] * 0.0
    return (out2, tc)


# speedup vs baseline: 2.8018x; 2.8018x over previous
import functools

import jax
import jax.numpy as jnp
from jax import lax
from jax.experimental import pallas as pl
from jax.experimental.pallas import tpu as pltpu
from jax.experimental.pallas import tpu_sc as plsc

_NWORKERS = 32
_CHUNK = 16384
_NBUF = 4


def _sc_copy_body(nchunk, src, dst, buf, *sems):
    sem_in = sems[:_NBUF]
    sem_out = sems[_NBUF:]
    per_w = nchunk * _CHUNK
    wid = lax.axis_index("s") * 2 + lax.axis_index("c")
    base = wid * per_w

    def start_load(b, off):
        pltpu.make_async_copy(
            src.at[pl.ds(off, _CHUNK)], buf.at[b], sem_in[b]).start()

    def wait_load(b):
        pltpu.make_async_copy(
            src.at[pl.ds(0, _CHUNK)], buf.at[b], sem_in[b]).wait()

    def start_store(b, off):
        pltpu.make_async_copy(
            buf.at[b], dst.at[pl.ds(off, _CHUNK)], sem_out[b]).start()

    def wait_store(b):
        pltpu.make_async_copy(
            buf.at[0], dst.at[pl.ds(0, _CHUNK)], sem_out[b]).wait()

    for b in range(_NBUF):
        start_load(b, base + b * _CHUNK)

    niter = nchunk // _NBUF

    def body(i, _):
        for b in range(_NBUF):
            wait_load(b)
            start_store(b, base + (i * _NBUF + b) * _CHUNK)
        for b in range(_NBUF):
            wait_store(b)

            @pl.when(i < niter - 1)
            def _():
                start_load(b, base + ((i + 1) * _NBUF + b) * _CHUNK)

        return 0

    lax.fori_loop(0, niter, body, 0)


def _sc_copy(x):
    n = x.size
    per_w = n // _NWORKERS
    nchunk = per_w // _CHUNK
    assert per_w % _CHUNK == 0 and nchunk % _NBUF == 0
    flat = x.reshape(n)
    mesh = plsc.VectorSubcoreMesh(core_axis_name="c", subcore_axis_name="s")
    snap = pl.kernel(
        functools.partial(_sc_copy_body, nchunk),
        out_type=jax.ShapeDtypeStruct((n,), x.dtype),
        mesh=mesh,
        scratch_types=(
            [pltpu.VMEM((_NBUF, _CHUNK), x.dtype)]
            + [pltpu.SemaphoreType.DMA] * (2 * _NBUF)
        ),
    )(flat)
    return snap.reshape(x.shape)


def _tc_copy_body(src_ref, dst_ref):
    dst_ref[...] = src_ref[...]


def _tc_copy(bank):
    dim, size = bank.shape
    blk = 16384
    return pl.pallas_call(
        _tc_copy_body,
        grid=(size // blk,),
        in_specs=[pl.BlockSpec((dim, blk), lambda i: (0, i))],
        out_specs=pl.BlockSpec((dim, blk), lambda i: (0, i)),
        out_shape=jax.ShapeDtypeStruct(bank.shape, bank.dtype),
    )(bank)


def kernel(output, bank):
    return (_sc_copy(output), _tc_copy(bank))
